# baseline (speedup 1.0000x reference)
import functools

import jax
import jax.numpy as jnp
from jax import lax
from jax.experimental import pallas as pl
from jax.experimental.pallas import tpu as pltpu
from jax.experimental.pallas import tpu_sc as plsc

_NC = 2
_NS = 16
_NW = _NC * _NS

_CHUNK = 32
_LANES = 16
_STREAM_FRAC = 0.42


@functools.partial(jax.jit, static_argnums=(2, 3))
def _sc_gather(idx, table, b_per_w, d):
    b = idx.shape[0]
    v = table.shape[0]
    n_chunks = b_per_w // _CHUNK
    n_seg = d // _LANES
    a = int(n_chunks * _STREAM_FRAC) // 2 * 2
    nt = n_chunks - a
    ni = max(a, nt)
    assert ni % 2 == 0 and nt % 2 == 0
    mesh = plsc.VectorSubcoreMesh(core_axis_name="c", subcore_axis_name="s")

    @functools.partial(
        pl.kernel,
        out_type=jax.ShapeDtypeStruct((b, d), jnp.float32),
        mesh=mesh,
        scratch_types=[
            pltpu.VMEM((v, d), jnp.float32),
            pltpu.VMEM((b_per_w,), jnp.int32),
            pltpu.VMEM((_CHUNK, d), jnp.float32),
            pltpu.VMEM((_CHUNK, d), jnp.float32),
            pltpu.VMEM((_CHUNK, d), jnp.float32),
            pltpu.VMEM((_CHUNK, d), jnp.float32),
            pltpu.SemaphoreType.DMA,
            pltpu.SemaphoreType.DMA,
            pltpu.SemaphoreType.DMA,
            pltpu.SemaphoreType.DMA,
            pltpu.SemaphoreType.DMA,
            pltpu.SemaphoreType.DMA,
        ],
    )
    def k(idx_hbm, table_hbm, out_hbm, table_v, idx_v, sb0, sb1, tb0, tb1,
          gs0, gs1, ws0, ws1, wt0, wt1):
        wid = lax.axis_index("s") * _NC + lax.axis_index("c")
        base = wid * b_per_w
        sb = (sb0, sb1)
        tb = (tb0, tb1)
        gsem = (gs0, gs1)
        wssem = (ws0, ws1)
        wtsem = (wt0, wt1)

        pltpu.sync_copy(table_hbm, table_v)
        pltpu.sync_copy(idx_hbm.at[pl.ds(base, b_per_w)], idx_v)

        def g_copy(i, bi):
            return pltpu.make_async_copy(
                table_hbm.at[idx_v.at[pl.ds(i * _CHUNK, _CHUNK)]],
                sb[bi],
                gsem[bi],
            )

        def ws_copy(i, bi):
            return pltpu.make_async_copy(
                sb[bi],
                out_hbm.at[pl.ds(base + i * _CHUNK, _CHUNK)],
                wssem[bi],
            )

        def wt_copy(i, bi):
            return pltpu.make_async_copy(
                tb[bi],
                out_hbm.at[pl.ds(base + (a + i) * _CHUNK, _CHUNK)],
                wtsem[bi],
            )

        def fill(i, bi):
            @plsc.parallel_loop(0, _CHUNK // _LANES)
            def _(jj):
                j0 = jj * _LANES
                rvec = idx_v[pl.ds((a + i) * _CHUNK + j0, _LANES)]
                for j in range(_LANES):
                    r = rvec[j]
                    for s in range(n_seg):
                        tb[bi][j0 + j, pl.ds(s * _LANES, _LANES)] = table_v[
                            r, pl.ds(s * _LANES, _LANES)
                        ]

        if a >= 1:
            g_copy(0, 0).start()

        def step(i, bi):
            @pl.when(i < nt)
            def _():
                @pl.when(i >= 2)
                def _():
                    wt_copy(i - 2, bi).wait()

                fill(i, bi)
                wt_copy(i, bi).start()

            @pl.when(i < a)
            def _():
                @pl.when(i >= 1)
                def _():
                    ws_copy(i - 1, bi ^ 1).wait()

                @pl.when(i + 1 < a)
                def _():
                    g_copy(i + 1, bi ^ 1).start()

                g_copy(i, bi).wait()
                ws_copy(i, bi).start()

        def body(grp, _):
            i = grp * 2
            step(i, 0)
            step(i + 1, 1)
            return 0

        lax.fori_loop(0, ni // 2, body, 0)
        if a >= 1:
            ws_copy(a - 1, (a - 1) % 2).wait()
        wt_copy(nt - 2, 0).wait()
        wt_copy(nt - 1, 1).wait()

    return k(idx, table)


def kernel(text, max_seq_len, letter_embedding, positional_encoding):
    del max_seq_len, positional_encoding
    b0, s = text.shape
    v, d = letter_embedding.shape
    b = b0 * s
    idx = text.reshape(b).astype(jnp.int32)
    b_per_w = b // _NW
    out = _sc_gather(idx, letter_embedding, b_per_w, d)
    return out.reshape(b0, s, d)


# speedup vs baseline: 1.1083x; 1.1083x over previous
import functools

import jax
import jax.numpy as jnp
from jax import lax
from jax.experimental import pallas as pl
from jax.experimental.pallas import tpu as pltpu
from jax.experimental.pallas import tpu_sc as plsc

_NC = 2
_NS = 16
_NW = _NC * _NS

_CHUNK = 32
_LANES = 16
_STREAM_FRAC = 0.52


@functools.partial(jax.jit, static_argnums=(2, 3))
def _sc_gather(idx, table, b_per_w, d):
    b = idx.shape[0]
    v = table.shape[0]
    n_chunks = b_per_w // _CHUNK
    n_seg = d // _LANES
    a = int(n_chunks * _STREAM_FRAC) // 2 * 2
    nt = n_chunks - a
    ni = max(a, nt)
    assert ni % 2 == 0 and nt % 2 == 0
    mesh = plsc.VectorSubcoreMesh(core_axis_name="c", subcore_axis_name="s")

    @functools.partial(
        pl.kernel,
        out_type=jax.ShapeDtypeStruct((b, d), jnp.float32),
        mesh=mesh,
        scratch_types=[
            pltpu.VMEM((v, d), jnp.float32),
            pltpu.VMEM((b_per_w,), jnp.int32),
            pltpu.VMEM((_CHUNK, d), jnp.float32),
            pltpu.VMEM((_CHUNK, d), jnp.float32),
            pltpu.VMEM((_CHUNK, d), jnp.float32),
            pltpu.VMEM((_CHUNK, d), jnp.float32),
            pltpu.SemaphoreType.DMA,
            pltpu.SemaphoreType.DMA,
            pltpu.SemaphoreType.DMA,
            pltpu.SemaphoreType.DMA,
            pltpu.SemaphoreType.DMA,
            pltpu.SemaphoreType.DMA,
        ],
    )
    def k(idx_hbm, table_hbm, out_hbm, table_v, idx_v, sb0, sb1, tb0, tb1,
          gs0, gs1, ws0, ws1, wt0, wt1):
        wid = lax.axis_index("s") * _NC + lax.axis_index("c")
        base = wid * b_per_w
        sb = (sb0, sb1)
        tb = (tb0, tb1)
        gsem = (gs0, gs1)
        wssem = (ws0, ws1)
        wtsem = (wt0, wt1)

        pltpu.sync_copy(table_hbm, table_v)
        pltpu.sync_copy(idx_hbm.at[pl.ds(base, b_per_w)], idx_v)

        def g_copy(i, bi):
            return pltpu.make_async_copy(
                table_hbm.at[idx_v.at[pl.ds(i * _CHUNK, _CHUNK)]],
                sb[bi],
                gsem[bi],
            )

        def ws_copy(i, bi):
            return pltpu.make_async_copy(
                sb[bi],
                out_hbm.at[pl.ds(base + i * _CHUNK, _CHUNK)],
                wssem[bi],
            )

        def wt_copy(i, bi):
            return pltpu.make_async_copy(
                tb[bi],
                out_hbm.at[pl.ds(base + (a + i) * _CHUNK, _CHUNK)],
                wtsem[bi],
            )

        def fill(i, bi):
            @plsc.parallel_loop(0, _CHUNK // _LANES)
            def _(jj):
                j0 = jj * _LANES
                rvec = idx_v[pl.ds((a + i) * _CHUNK + j0, _LANES)]
                for j in range(_LANES):
                    r = rvec[j]
                    for s in range(n_seg):
                        tb[bi][j0 + j, pl.ds(s * _LANES, _LANES)] = table_v[
                            r, pl.ds(s * _LANES, _LANES)
                        ]

        if a >= 1:
            g_copy(0, 0).start()

        def step(i, bi):
            @pl.when(i < nt)
            def _():
                @pl.when(i >= 2)
                def _():
                    wt_copy(i - 2, bi).wait()

                fill(i, bi)
                wt_copy(i, bi).start()

            @pl.when(i < a)
            def _():
                @pl.when(i >= 1)
                def _():
                    ws_copy(i - 1, bi ^ 1).wait()

                @pl.when(i + 1 < a)
                def _():
                    g_copy(i + 1, bi ^ 1).start()

                g_copy(i, bi).wait()
                ws_copy(i, bi).start()

        def body(grp, _):
            i = grp * 2
            step(i, 0)
            step(i + 1, 1)
            return 0

        lax.fori_loop(0, ni // 2, body, 0)
        if a >= 1:
            ws_copy(a - 1, (a - 1) % 2).wait()
        wt_copy(nt - 2, 0).wait()
        wt_copy(nt - 1, 1).wait()

    return k(idx, table)


def kernel(text, max_seq_len, letter_embedding, positional_encoding):
    del max_seq_len, positional_encoding
    b0, s = text.shape
    v, d = letter_embedding.shape
    b = b0 * s
    idx = text.reshape(b).astype(jnp.int32)
    b_per_w = b // _NW
    out = _sc_gather(idx, letter_embedding, b_per_w, d)
    return out.reshape(b0, s, d)


# speedup vs baseline: 1.1842x; 1.0686x over previous
import functools

import jax
import jax.numpy as jnp
from jax import lax
from jax.experimental import pallas as pl
from jax.experimental.pallas import tpu as pltpu
from jax.experimental.pallas import tpu_sc as plsc

_NC = 2
_NS = 16
_NW = _NC * _NS

_CHUNK = 32
_LANES = 16
_STREAM_FRAC = 0.52


@functools.partial(jax.jit, static_argnums=(2, 3))
def _sc_gather(idx, table, b_per_w, d):
    b = idx.shape[0]
    v = table.shape[0]
    n_chunks = b_per_w // _CHUNK
    n_seg = d // _LANES
    a = int(n_chunks * _STREAM_FRAC) // 2 * 2
    nt = n_chunks - a
    ni = max(a, nt)
    assert ni % 2 == 0 and nt % 2 == 0
    mesh = plsc.VectorSubcoreMesh(core_axis_name="c", subcore_axis_name="s")

    @functools.partial(
        pl.kernel,
        out_type=jax.ShapeDtypeStruct((b, d), jnp.float32),
        mesh=mesh,
        scratch_types=[
            pltpu.VMEM((v, d), jnp.float32),
            pltpu.VMEM((b_per_w,), jnp.int32),
            pltpu.VMEM((_CHUNK, d), jnp.float32),
            pltpu.VMEM((_CHUNK, d), jnp.float32),
            pltpu.VMEM((_CHUNK, d), jnp.float32),
            pltpu.VMEM((_CHUNK, d), jnp.float32),
            pltpu.SemaphoreType.DMA,
            pltpu.SemaphoreType.DMA,
            pltpu.SemaphoreType.DMA,
            pltpu.SemaphoreType.DMA,
            pltpu.SemaphoreType.DMA,
            pltpu.SemaphoreType.DMA,
        ],
    )
    def k(idx_hbm, table_hbm, out_hbm, table_v, idx_v, sb0, sb1, tb0, tb1,
          gs0, gs1, ws0, ws1, wt0, wt1):
        wid = lax.axis_index("s") * _NC + lax.axis_index("c")
        base = wid * b_per_w
        sb = (sb0, sb1)
        tb = (tb0, tb1)
        gsem = (gs0, gs1)
        wssem = (ws0, ws1)
        wtsem = (wt0, wt1)

        pltpu.sync_copy(table_hbm, table_v)
        pltpu.sync_copy(idx_hbm.at[pl.ds(base, b_per_w)], idx_v)

        def g_copy(i, bi):
            return pltpu.make_async_copy(
                table_hbm.at[idx_v.at[pl.ds(i * _CHUNK, _CHUNK)]],
                sb[bi],
                gsem[bi],
            )

        def ws_copy(i, bi):
            return pltpu.make_async_copy(
                sb[bi],
                out_hbm.at[pl.ds(base + i * _CHUNK, _CHUNK)],
                wssem[bi],
            )

        def wt_copy(i, bi):
            return pltpu.make_async_copy(
                tb[bi],
                out_hbm.at[pl.ds(base + (a + i) * _CHUNK, _CHUNK)],
                wtsem[bi],
            )

        def fill(i, bi):
            @plsc.parallel_loop(0, _CHUNK // _LANES)
            def _(jj):
                j0 = jj * _LANES
                rvec = idx_v[pl.ds((a + i) * _CHUNK + j0, _LANES)]
                for j in range(_LANES):
                    r = rvec[j]
                    for s0 in range(0, n_seg, 4):
                        vals = [
                            table_v[r, pl.ds((s0 + s) * _LANES, _LANES)]
                            for s in range(4)
                        ]
                        for s in range(4):
                            tb[bi][
                                j0 + j, pl.ds((s0 + s) * _LANES, _LANES)
                            ] = vals[s]

        if a >= 1:
            g_copy(0, 0).start()

        def step(i, bi):
            @pl.when(i < nt)
            def _():
                @pl.when(i >= 2)
                def _():
                    wt_copy(i - 2, bi).wait()

                fill(i, bi)
                wt_copy(i, bi).start()

            @pl.when(i < a)
            def _():
                @pl.when(i >= 1)
                def _():
                    ws_copy(i - 1, bi ^ 1).wait()

                @pl.when(i + 1 < a)
                def _():
                    g_copy(i + 1, bi ^ 1).start()

                g_copy(i, bi).wait()
                ws_copy(i, bi).start()

        def body(grp, _):
            i = grp * 2
            step(i, 0)
            step(i + 1, 1)
            return 0

        lax.fori_loop(0, ni // 2, body, 0)
        if a >= 1:
            ws_copy(a - 1, (a - 1) % 2).wait()
        wt_copy(nt - 2, 0).wait()
        wt_copy(nt - 1, 1).wait()

    return k(idx, table)


def kernel(text, max_seq_len, letter_embedding, positional_encoding):
    del max_seq_len, positional_encoding
    b0, s = text.shape
    v, d = letter_embedding.shape
    b = b0 * s
    idx = text.reshape(b).astype(jnp.int32)
    b_per_w = b // _NW
    out = _sc_gather(idx, letter_embedding, b_per_w, d)
    return out.reshape(b0, s, d)


# speedup vs baseline: 1.2288x; 1.0376x over previous
import functools

import jax
import jax.numpy as jnp
from jax import lax
from jax.experimental import pallas as pl
from jax.experimental.pallas import tpu as pltpu
from jax.experimental.pallas import tpu_sc as plsc

_NC = 2
_NS = 16
_NW = _NC * _NS

_CHUNK = 32
_LANES = 16
_STREAM_FRAC = 0.46


@functools.partial(jax.jit, static_argnums=(2, 3))
def _sc_gather(idx, table, b_per_w, d):
    b = idx.shape[0]
    v = table.shape[0]
    n_chunks = b_per_w // _CHUNK
    n_seg = d // _LANES
    a = int(n_chunks * _STREAM_FRAC) // 2 * 2
    nt = n_chunks - a
    ni = max(a, nt)
    assert ni % 2 == 0 and nt % 2 == 0
    mesh = plsc.VectorSubcoreMesh(core_axis_name="c", subcore_axis_name="s")

    @functools.partial(
        pl.kernel,
        out_type=jax.ShapeDtypeStruct((b, d), jnp.float32),
        mesh=mesh,
        scratch_types=[
            pltpu.VMEM((v, d), jnp.float32),
            pltpu.VMEM((b_per_w,), jnp.int32),
            pltpu.VMEM((_CHUNK, d), jnp.float32),
            pltpu.VMEM((_CHUNK, d), jnp.float32),
            pltpu.VMEM((_CHUNK, d), jnp.float32),
            pltpu.VMEM((_CHUNK, d), jnp.float32),
            pltpu.SemaphoreType.DMA,
            pltpu.SemaphoreType.DMA,
            pltpu.SemaphoreType.DMA,
            pltpu.SemaphoreType.DMA,
            pltpu.SemaphoreType.DMA,
            pltpu.SemaphoreType.DMA,
        ],
    )
    def k(idx_hbm, table_hbm, out_hbm, table_v, idx_v, sb0, sb1, tb0, tb1,
          gs0, gs1, ws0, ws1, wt0, wt1):
        wid = lax.axis_index("s") * _NC + lax.axis_index("c")
        base = wid * b_per_w
        sb = (sb0, sb1)
        tb = (tb0, tb1)
        gsem = (gs0, gs1)
        wssem = (ws0, ws1)
        wtsem = (wt0, wt1)

        pltpu.sync_copy(table_hbm, table_v)
        pltpu.sync_copy(idx_hbm.at[pl.ds(base, b_per_w)], idx_v)

        def g_copy(i, bi):
            return pltpu.make_async_copy(
                table_hbm.at[idx_v.at[pl.ds(i * _CHUNK, _CHUNK)]],
                sb[bi],
                gsem[bi],
            )

        def ws_copy(i, bi):
            return pltpu.make_async_copy(
                sb[bi],
                out_hbm.at[pl.ds(base + i * _CHUNK, _CHUNK)],
                wssem[bi],
            )

        def wt_copy(i, bi):
            return pltpu.make_async_copy(
                tb[bi],
                out_hbm.at[pl.ds(base + (a + i) * _CHUNK, _CHUNK)],
                wtsem[bi],
            )

        def fill(i, bi):
            @plsc.parallel_loop(0, _CHUNK // _LANES)
            def _(jj):
                j0 = jj * _LANES
                rvec = idx_v[pl.ds((a + i) * _CHUNK + j0, _LANES)]
                for j in range(_LANES):
                    r = rvec[j]
                    for s0 in range(0, n_seg, 4):
                        vals = [
                            table_v[r, pl.ds((s0 + s) * _LANES, _LANES)]
                            for s in range(4)
                        ]
                        for s in range(4):
                            tb[bi][
                                j0 + j, pl.ds((s0 + s) * _LANES, _LANES)
                            ] = vals[s]

        if a >= 1:
            g_copy(0, 0).start()

        def step(i, bi):
            @pl.when(i < nt)
            def _():
                @pl.when(i >= 2)
                def _():
                    wt_copy(i - 2, bi).wait()

                fill(i, bi)
                wt_copy(i, bi).start()

            @pl.when(i < a)
            def _():
                @pl.when(i >= 1)
                def _():
                    ws_copy(i - 1, bi ^ 1).wait()

                @pl.when(i + 1 < a)
                def _():
                    g_copy(i + 1, bi ^ 1).start()

                g_copy(i, bi).wait()
                ws_copy(i, bi).start()

        def body(grp, _):
            i = grp * 2
            step(i, 0)
            step(i + 1, 1)
            return 0

        lax.fori_loop(0, ni // 2, body, 0)
        if a >= 1:
            ws_copy(a - 1, (a - 1) % 2).wait()
        wt_copy(nt - 2, 0).wait()
        wt_copy(nt - 1, 1).wait()

    return k(idx, table)


def kernel(text, max_seq_len, letter_embedding, positional_encoding):
    del max_seq_len, positional_encoding
    b0, s = text.shape
    v, d = letter_embedding.shape
    b = b0 * s
    idx = text.reshape(b).astype(jnp.int32)
    b_per_w = b // _NW
    out = _sc_gather(idx, letter_embedding, b_per_w, d)
    return out.reshape(b0, s, d)


# speedup vs baseline: 1.2639x; 1.0285x over previous
import functools

import jax
import jax.numpy as jnp
from jax import lax
from jax.experimental import pallas as pl
from jax.experimental.pallas import tpu as pltpu
from jax.experimental.pallas import tpu_sc as plsc

_NC = 2
_NS = 16
_NW = _NC * _NS

_CHUNK = 32
_LANES = 16
_STREAM_FRAC = 0.40


@functools.partial(jax.jit, static_argnums=(2, 3))
def _sc_gather(idx, table, b_per_w, d):
    b = idx.shape[0]
    v = table.shape[0]
    n_chunks = b_per_w // _CHUNK
    n_seg = d // _LANES
    a = int(n_chunks * _STREAM_FRAC) // 2 * 2
    nt = n_chunks - a
    ni = max(a, nt)
    assert ni % 2 == 0 and nt % 2 == 0
    mesh = plsc.VectorSubcoreMesh(core_axis_name="c", subcore_axis_name="s")

    @functools.partial(
        pl.kernel,
        out_type=jax.ShapeDtypeStruct((b, d), jnp.float32),
        mesh=mesh,
        scratch_types=[
            pltpu.VMEM((v, d), jnp.float32),
            pltpu.VMEM((b_per_w,), jnp.int32),
            pltpu.VMEM((_CHUNK, d), jnp.float32),
            pltpu.VMEM((_CHUNK, d), jnp.float32),
            pltpu.VMEM((_CHUNK, d), jnp.float32),
            pltpu.VMEM((_CHUNK, d), jnp.float32),
            pltpu.SemaphoreType.DMA,
            pltpu.SemaphoreType.DMA,
            pltpu.SemaphoreType.DMA,
            pltpu.SemaphoreType.DMA,
            pltpu.SemaphoreType.DMA,
            pltpu.SemaphoreType.DMA,
        ],
    )
    def k(idx_hbm, table_hbm, out_hbm, table_v, idx_v, sb0, sb1, tb0, tb1,
          gs0, gs1, ws0, ws1, wt0, wt1):
        wid = lax.axis_index("s") * _NC + lax.axis_index("c")
        base = wid * b_per_w
        sb = (sb0, sb1)
        tb = (tb0, tb1)
        gsem = (gs0, gs1)
        wssem = (ws0, ws1)
        wtsem = (wt0, wt1)

        pltpu.sync_copy(table_hbm, table_v)
        pltpu.sync_copy(idx_hbm.at[pl.ds(base, b_per_w)], idx_v)

        def g_copy(i, bi):
            return pltpu.make_async_copy(
                table_hbm.at[idx_v.at[pl.ds(i * _CHUNK, _CHUNK)]],
                sb[bi],
                gsem[bi],
            )

        def ws_copy(i, bi):
            return pltpu.make_async_copy(
                sb[bi],
                out_hbm.at[pl.ds(base + i * _CHUNK, _CHUNK)],
                wssem[bi],
            )

        def wt_copy(i, bi):
            return pltpu.make_async_copy(
                tb[bi],
                out_hbm.at[pl.ds(base + (a + i) * _CHUNK, _CHUNK)],
                wtsem[bi],
            )

        def fill(i, bi):
            @plsc.parallel_loop(0, _CHUNK // _LANES)
            def _(jj):
                j0 = jj * _LANES
                rvec = idx_v[pl.ds((a + i) * _CHUNK + j0, _LANES)]
                for j in range(_LANES):
                    r = rvec[j]
                    for s0 in range(0, n_seg, 4):
                        vals = [
                            table_v[r, pl.ds((s0 + s) * _LANES, _LANES)]
                            for s in range(4)
                        ]
                        for s in range(4):
                            tb[bi][
                                j0 + j, pl.ds((s0 + s) * _LANES, _LANES)
                            ] = vals[s]

        if a >= 1:
            g_copy(0, 0).start()

        def step(i, bi):
            @pl.when(i < nt)
            def _():
                @pl.when(i >= 2)
                def _():
                    wt_copy(i - 2, bi).wait()

                fill(i, bi)
                wt_copy(i, bi).start()

            @pl.when(i < a)
            def _():
                @pl.when(i >= 1)
                def _():
                    ws_copy(i - 1, bi ^ 1).wait()

                @pl.when(i + 1 < a)
                def _():
                    g_copy(i + 1, bi ^ 1).start()

                g_copy(i, bi).wait()
                ws_copy(i, bi).start()

        def body(grp, _):
            i = grp * 2
            step(i, 0)
            step(i + 1, 1)
            return 0

        lax.fori_loop(0, ni // 2, body, 0)
        if a >= 1:
            ws_copy(a - 1, (a - 1) % 2).wait()
        wt_copy(nt - 2, 0).wait()
        wt_copy(nt - 1, 1).wait()

    return k(idx, table)


def kernel(text, max_seq_len, letter_embedding, positional_encoding):
    del max_seq_len, positional_encoding
    b0, s = text.shape
    v, d = letter_embedding.shape
    b = b0 * s
    idx = text.reshape(b).astype(jnp.int32)
    b_per_w = b // _NW
    out = _sc_gather(idx, letter_embedding, b_per_w, d)
    return out.reshape(b0, s, d)


# speedup vs baseline: 1.2941x; 1.0240x over previous
import functools

import jax
import jax.numpy as jnp
from jax import lax
from jax.experimental import pallas as pl
from jax.experimental.pallas import tpu as pltpu
from jax.experimental.pallas import tpu_sc as plsc

_NC = 2
_NS = 16
_NW = _NC * _NS

_CHUNK = 32
_LANES = 16
_STREAM_FRAC = 0.34


@functools.partial(jax.jit, static_argnums=(2, 3))
def _sc_gather(idx, table, b_per_w, d):
    b = idx.shape[0]
    v = table.shape[0]
    n_chunks = b_per_w // _CHUNK
    n_seg = d // _LANES
    a = int(n_chunks * _STREAM_FRAC) // 2 * 2
    nt = n_chunks - a
    ni = max(a, nt)
    assert ni % 2 == 0 and nt % 2 == 0
    mesh = plsc.VectorSubcoreMesh(core_axis_name="c", subcore_axis_name="s")

    @functools.partial(
        pl.kernel,
        out_type=jax.ShapeDtypeStruct((b, d), jnp.float32),
        mesh=mesh,
        scratch_types=[
            pltpu.VMEM((v, d), jnp.float32),
            pltpu.VMEM((b_per_w,), jnp.int32),
            pltpu.VMEM((_CHUNK, d), jnp.float32),
            pltpu.VMEM((_CHUNK, d), jnp.float32),
            pltpu.VMEM((_CHUNK, d), jnp.float32),
            pltpu.VMEM((_CHUNK, d), jnp.float32),
            pltpu.SemaphoreType.DMA,
            pltpu.SemaphoreType.DMA,
            pltpu.SemaphoreType.DMA,
            pltpu.SemaphoreType.DMA,
            pltpu.SemaphoreType.DMA,
            pltpu.SemaphoreType.DMA,
        ],
    )
    def k(idx_hbm, table_hbm, out_hbm, table_v, idx_v, sb0, sb1, tb0, tb1,
          gs0, gs1, ws0, ws1, wt0, wt1):
        wid = lax.axis_index("s") * _NC + lax.axis_index("c")
        base = wid * b_per_w
        sb = (sb0, sb1)
        tb = (tb0, tb1)
        gsem = (gs0, gs1)
        wssem = (ws0, ws1)
        wtsem = (wt0, wt1)

        pltpu.sync_copy(table_hbm, table_v)
        pltpu.sync_copy(idx_hbm.at[pl.ds(base, b_per_w)], idx_v)

        def g_copy(i, bi):
            return pltpu.make_async_copy(
                table_hbm.at[idx_v.at[pl.ds(i * _CHUNK, _CHUNK)]],
                sb[bi],
                gsem[bi],
            )

        def ws_copy(i, bi):
            return pltpu.make_async_copy(
                sb[bi],
                out_hbm.at[pl.ds(base + i * _CHUNK, _CHUNK)],
                wssem[bi],
            )

        def wt_copy(i, bi):
            return pltpu.make_async_copy(
                tb[bi],
                out_hbm.at[pl.ds(base + (a + i) * _CHUNK, _CHUNK)],
                wtsem[bi],
            )

        def fill(i, bi):
            @plsc.parallel_loop(0, _CHUNK // _LANES)
            def _(jj):
                j0 = jj * _LANES
                rvec = idx_v[pl.ds((a + i) * _CHUNK + j0, _LANES)]
                for j in range(_LANES):
                    r = rvec[j]
                    for s0 in range(0, n_seg, 4):
                        vals = [
                            table_v[r, pl.ds((s0 + s) * _LANES, _LANES)]
                            for s in range(4)
                        ]
                        for s in range(4):
                            tb[bi][
                                j0 + j, pl.ds((s0 + s) * _LANES, _LANES)
                            ] = vals[s]

        if a >= 1:
            g_copy(0, 0).start()

        def step(i, bi):
            @pl.when(i < nt)
            def _():
                @pl.when(i >= 2)
                def _():
                    wt_copy(i - 2, bi).wait()

                fill(i, bi)
                wt_copy(i, bi).start()

            @pl.when(i < a)
            def _():
                @pl.when(i >= 1)
                def _():
                    ws_copy(i - 1, bi ^ 1).wait()

                @pl.when(i + 1 < a)
                def _():
                    g_copy(i + 1, bi ^ 1).start()

                g_copy(i, bi).wait()
                ws_copy(i, bi).start()

        def body(grp, _):
            i = grp * 2
            step(i, 0)
            step(i + 1, 1)
            return 0

        lax.fori_loop(0, ni // 2, body, 0)
        if a >= 1:
            ws_copy(a - 1, (a - 1) % 2).wait()
        wt_copy(nt - 2, 0).wait()
        wt_copy(nt - 1, 1).wait()

    return k(idx, table)


def kernel(text, max_seq_len, letter_embedding, positional_encoding):
    del max_seq_len, positional_encoding
    b0, s = text.shape
    v, d = letter_embedding.shape
    b = b0 * s
    idx = text.reshape(b).astype(jnp.int32)
    b_per_w = b // _NW
    out = _sc_gather(idx, letter_embedding, b_per_w, d)
    return out.reshape(b0, s, d)


# speedup vs baseline: 1.3443x; 1.0388x over previous
import functools

import jax
import jax.numpy as jnp
from jax import lax
from jax.experimental import pallas as pl
from jax.experimental.pallas import tpu as pltpu
from jax.experimental.pallas import tpu_sc as plsc

_NC = 2
_NS = 16
_NW = _NC * _NS

_CHUNK = 32
_LANES = 16
_STREAM_FRAC = 0.26


@functools.partial(jax.jit, static_argnums=(2, 3))
def _sc_gather(idx, table, b_per_w, d):
    b = idx.shape[0]
    v = table.shape[0]
    n_chunks = b_per_w // _CHUNK
    n_seg = d // _LANES
    a = int(n_chunks * _STREAM_FRAC) // 2 * 2
    nt = n_chunks - a
    ni = max(a, nt)
    assert ni % 2 == 0 and nt % 2 == 0
    mesh = plsc.VectorSubcoreMesh(core_axis_name="c", subcore_axis_name="s")

    @functools.partial(
        pl.kernel,
        out_type=jax.ShapeDtypeStruct((b, d), jnp.float32),
        mesh=mesh,
        scratch_types=[
            pltpu.VMEM((v, d), jnp.float32),
            pltpu.VMEM((b_per_w,), jnp.int32),
            pltpu.VMEM((_CHUNK, d), jnp.float32),
            pltpu.VMEM((_CHUNK, d), jnp.float32),
            pltpu.VMEM((_CHUNK, d), jnp.float32),
            pltpu.VMEM((_CHUNK, d), jnp.float32),
            pltpu.SemaphoreType.DMA,
            pltpu.SemaphoreType.DMA,
            pltpu.SemaphoreType.DMA,
            pltpu.SemaphoreType.DMA,
            pltpu.SemaphoreType.DMA,
            pltpu.SemaphoreType.DMA,
        ],
    )
    def k(idx_hbm, table_hbm, out_hbm, table_v, idx_v, sb0, sb1, tb0, tb1,
          gs0, gs1, ws0, ws1, wt0, wt1):
        wid = lax.axis_index("s") * _NC + lax.axis_index("c")
        base = wid * b_per_w
        sb = (sb0, sb1)
        tb = (tb0, tb1)
        gsem = (gs0, gs1)
        wssem = (ws0, ws1)
        wtsem = (wt0, wt1)

        pltpu.sync_copy(table_hbm, table_v)
        pltpu.sync_copy(idx_hbm.at[pl.ds(base, b_per_w)], idx_v)

        def g_copy(i, bi):
            return pltpu.make_async_copy(
                table_hbm.at[idx_v.at[pl.ds(i * _CHUNK, _CHUNK)]],
                sb[bi],
                gsem[bi],
            )

        def ws_copy(i, bi):
            return pltpu.make_async_copy(
                sb[bi],
                out_hbm.at[pl.ds(base + i * _CHUNK, _CHUNK)],
                wssem[bi],
            )

        def wt_copy(i, bi):
            return pltpu.make_async_copy(
                tb[bi],
                out_hbm.at[pl.ds(base + (a + i) * _CHUNK, _CHUNK)],
                wtsem[bi],
            )

        def fill(i, bi):
            @plsc.parallel_loop(0, _CHUNK // _LANES)
            def _(jj):
                j0 = jj * _LANES
                rvec = idx_v[pl.ds((a + i) * _CHUNK + j0, _LANES)]
                for j in range(_LANES):
                    r = rvec[j]
                    for s0 in range(0, n_seg, 4):
                        vals = [
                            table_v[r, pl.ds((s0 + s) * _LANES, _LANES)]
                            for s in range(4)
                        ]
                        for s in range(4):
                            tb[bi][
                                j0 + j, pl.ds((s0 + s) * _LANES, _LANES)
                            ] = vals[s]

        if a >= 1:
            g_copy(0, 0).start()

        def step(i, bi):
            @pl.when(i < nt)
            def _():
                @pl.when(i >= 2)
                def _():
                    wt_copy(i - 2, bi).wait()

                fill(i, bi)
                wt_copy(i, bi).start()

            @pl.when(i < a)
            def _():
                @pl.when(i >= 1)
                def _():
                    ws_copy(i - 1, bi ^ 1).wait()

                @pl.when(i + 1 < a)
                def _():
                    g_copy(i + 1, bi ^ 1).start()

                g_copy(i, bi).wait()
                ws_copy(i, bi).start()

        def body(grp, _):
            i = grp * 2
            step(i, 0)
            step(i + 1, 1)
            return 0

        lax.fori_loop(0, ni // 2, body, 0)
        if a >= 1:
            ws_copy(a - 1, (a - 1) % 2).wait()
        wt_copy(nt - 2, 0).wait()
        wt_copy(nt - 1, 1).wait()

    return k(idx, table)


def kernel(text, max_seq_len, letter_embedding, positional_encoding):
    del max_seq_len, positional_encoding
    b0, s = text.shape
    v, d = letter_embedding.shape
    b = b0 * s
    idx = text.reshape(b).astype(jnp.int32)
    b_per_w = b // _NW
    out = _sc_gather(idx, letter_embedding, b_per_w, d)
    return out.reshape(b0, s, d)


# speedup vs baseline: 1.3826x; 1.0285x over previous
import functools

import jax
import jax.numpy as jnp
from jax import lax
from jax.experimental import pallas as pl
from jax.experimental.pallas import tpu as pltpu
from jax.experimental.pallas import tpu_sc as plsc

_NC = 2
_NS = 16
_NW = _NC * _NS

_CHUNK = 32
_LANES = 16
_STREAM_FRAC = 0.18


@functools.partial(jax.jit, static_argnums=(2, 3))
def _sc_gather(idx, table, b_per_w, d):
    b = idx.shape[0]
    v = table.shape[0]
    n_chunks = b_per_w // _CHUNK
    n_seg = d // _LANES
    a = int(n_chunks * _STREAM_FRAC) // 2 * 2
    nt = n_chunks - a
    ni = max(a, nt)
    assert ni % 2 == 0 and nt % 2 == 0
    mesh = plsc.VectorSubcoreMesh(core_axis_name="c", subcore_axis_name="s")

    @functools.partial(
        pl.kernel,
        out_type=jax.ShapeDtypeStruct((b, d), jnp.float32),
        mesh=mesh,
        scratch_types=[
            pltpu.VMEM((v, d), jnp.float32),
            pltpu.VMEM((b_per_w,), jnp.int32),
            pltpu.VMEM((_CHUNK, d), jnp.float32),
            pltpu.VMEM((_CHUNK, d), jnp.float32),
            pltpu.VMEM((_CHUNK, d), jnp.float32),
            pltpu.VMEM((_CHUNK, d), jnp.float32),
            pltpu.SemaphoreType.DMA,
            pltpu.SemaphoreType.DMA,
            pltpu.SemaphoreType.DMA,
            pltpu.SemaphoreType.DMA,
            pltpu.SemaphoreType.DMA,
            pltpu.SemaphoreType.DMA,
        ],
    )
    def k(idx_hbm, table_hbm, out_hbm, table_v, idx_v, sb0, sb1, tb0, tb1,
          gs0, gs1, ws0, ws1, wt0, wt1):
        wid = lax.axis_index("s") * _NC + lax.axis_index("c")
        base = wid * b_per_w
        sb = (sb0, sb1)
        tb = (tb0, tb1)
        gsem = (gs0, gs1)
        wssem = (ws0, ws1)
        wtsem = (wt0, wt1)

        pltpu.sync_copy(table_hbm, table_v)
        pltpu.sync_copy(idx_hbm.at[pl.ds(base, b_per_w)], idx_v)

        def g_copy(i, bi):
            return pltpu.make_async_copy(
                table_hbm.at[idx_v.at[pl.ds(i * _CHUNK, _CHUNK)]],
                sb[bi],
                gsem[bi],
            )

        def ws_copy(i, bi):
            return pltpu.make_async_copy(
                sb[bi],
                out_hbm.at[pl.ds(base + i * _CHUNK, _CHUNK)],
                wssem[bi],
            )

        def wt_copy(i, bi):
            return pltpu.make_async_copy(
                tb[bi],
                out_hbm.at[pl.ds(base + (a + i) * _CHUNK, _CHUNK)],
                wtsem[bi],
            )

        def fill(i, bi):
            @plsc.parallel_loop(0, _CHUNK // _LANES)
            def _(jj):
                j0 = jj * _LANES
                rvec = idx_v[pl.ds((a + i) * _CHUNK + j0, _LANES)]
                for j in range(_LANES):
                    r = rvec[j]
                    for s0 in range(0, n_seg, 4):
                        vals = [
                            table_v[r, pl.ds((s0 + s) * _LANES, _LANES)]
                            for s in range(4)
                        ]
                        for s in range(4):
                            tb[bi][
                                j0 + j, pl.ds((s0 + s) * _LANES, _LANES)
                            ] = vals[s]

        if a >= 1:
            g_copy(0, 0).start()

        def step(i, bi):
            @pl.when(i < nt)
            def _():
                @pl.when(i >= 2)
                def _():
                    wt_copy(i - 2, bi).wait()

                fill(i, bi)
                wt_copy(i, bi).start()

            @pl.when(i < a)
            def _():
                @pl.when(i >= 1)
                def _():
                    ws_copy(i - 1, bi ^ 1).wait()

                @pl.when(i + 1 < a)
                def _():
                    g_copy(i + 1, bi ^ 1).start()

                g_copy(i, bi).wait()
                ws_copy(i, bi).start()

        def body(grp, _):
            i = grp * 2
            step(i, 0)
            step(i + 1, 1)
            return 0

        lax.fori_loop(0, ni // 2, body, 0)
        if a >= 1:
            ws_copy(a - 1, (a - 1) % 2).wait()
        wt_copy(nt - 2, 0).wait()
        wt_copy(nt - 1, 1).wait()

    return k(idx, table)


def kernel(text, max_seq_len, letter_embedding, positional_encoding):
    del max_seq_len, positional_encoding
    b0, s = text.shape
    v, d = letter_embedding.shape
    b = b0 * s
    idx = text.reshape(b).astype(jnp.int32)
    b_per_w = b // _NW
    out = _sc_gather(idx, letter_embedding, b_per_w, d)
    return out.reshape(b0, s, d)


# speedup vs baseline: 1.4491x; 1.0481x over previous
import functools

import jax
import jax.numpy as jnp
from jax import lax
from jax.experimental import pallas as pl
from jax.experimental.pallas import tpu as pltpu
from jax.experimental.pallas import tpu_sc as plsc

_NC = 2
_NS = 16
_NW = _NC * _NS

_CHUNK = 32
_LANES = 16
_STREAM_FRAC = 0.10


@functools.partial(jax.jit, static_argnums=(2, 3))
def _sc_gather(idx, table, b_per_w, d):
    b = idx.shape[0]
    v = table.shape[0]
    n_chunks = b_per_w // _CHUNK
    n_seg = d // _LANES
    a = int(n_chunks * _STREAM_FRAC) // 2 * 2
    nt = n_chunks - a
    ni = max(a, nt)
    assert ni % 2 == 0 and nt % 2 == 0
    mesh = plsc.VectorSubcoreMesh(core_axis_name="c", subcore_axis_name="s")

    @functools.partial(
        pl.kernel,
        out_type=jax.ShapeDtypeStruct((b, d), jnp.float32),
        mesh=mesh,
        scratch_types=[
            pltpu.VMEM((v, d), jnp.float32),
            pltpu.VMEM((b_per_w,), jnp.int32),
            pltpu.VMEM((_CHUNK, d), jnp.float32),
            pltpu.VMEM((_CHUNK, d), jnp.float32),
            pltpu.VMEM((_CHUNK, d), jnp.float32),
            pltpu.VMEM((_CHUNK, d), jnp.float32),
            pltpu.SemaphoreType.DMA,
            pltpu.SemaphoreType.DMA,
            pltpu.SemaphoreType.DMA,
            pltpu.SemaphoreType.DMA,
            pltpu.SemaphoreType.DMA,
            pltpu.SemaphoreType.DMA,
        ],
    )
    def k(idx_hbm, table_hbm, out_hbm, table_v, idx_v, sb0, sb1, tb0, tb1,
          gs0, gs1, ws0, ws1, wt0, wt1):
        wid = lax.axis_index("s") * _NC + lax.axis_index("c")
        base = wid * b_per_w
        sb = (sb0, sb1)
        tb = (tb0, tb1)
        gsem = (gs0, gs1)
        wssem = (ws0, ws1)
        wtsem = (wt0, wt1)

        pltpu.sync_copy(table_hbm, table_v)
        pltpu.sync_copy(idx_hbm.at[pl.ds(base, b_per_w)], idx_v)

        def g_copy(i, bi):
            return pltpu.make_async_copy(
                table_hbm.at[idx_v.at[pl.ds(i * _CHUNK, _CHUNK)]],
                sb[bi],
                gsem[bi],
            )

        def ws_copy(i, bi):
            return pltpu.make_async_copy(
                sb[bi],
                out_hbm.at[pl.ds(base + i * _CHUNK, _CHUNK)],
                wssem[bi],
            )

        def wt_copy(i, bi):
            return pltpu.make_async_copy(
                tb[bi],
                out_hbm.at[pl.ds(base + (a + i) * _CHUNK, _CHUNK)],
                wtsem[bi],
            )

        def fill(i, bi):
            @plsc.parallel_loop(0, _CHUNK // _LANES)
            def _(jj):
                j0 = jj * _LANES
                rvec = idx_v[pl.ds((a + i) * _CHUNK + j0, _LANES)]
                for j in range(_LANES):
                    r = rvec[j]
                    for s0 in range(0, n_seg, 4):
                        vals = [
                            table_v[r, pl.ds((s0 + s) * _LANES, _LANES)]
                            for s in range(4)
                        ]
                        for s in range(4):
                            tb[bi][
                                j0 + j, pl.ds((s0 + s) * _LANES, _LANES)
                            ] = vals[s]

        if a >= 1:
            g_copy(0, 0).start()

        def step(i, bi):
            @pl.when(i < nt)
            def _():
                @pl.when(i >= 2)
                def _():
                    wt_copy(i - 2, bi).wait()

                fill(i, bi)
                wt_copy(i, bi).start()

            @pl.when(i < a)
            def _():
                @pl.when(i >= 1)
                def _():
                    ws_copy(i - 1, bi ^ 1).wait()

                @pl.when(i + 1 < a)
                def _():
                    g_copy(i + 1, bi ^ 1).start()

                g_copy(i, bi).wait()
                ws_copy(i, bi).start()

        def body(grp, _):
            i = grp * 2
            step(i, 0)
            step(i + 1, 1)
            return 0

        lax.fori_loop(0, ni // 2, body, 0)
        if a >= 1:
            ws_copy(a - 1, (a - 1) % 2).wait()
        wt_copy(nt - 2, 0).wait()
        wt_copy(nt - 1, 1).wait()

    return k(idx, table)


def kernel(text, max_seq_len, letter_embedding, positional_encoding):
    del max_seq_len, positional_encoding
    b0, s = text.shape
    v, d = letter_embedding.shape
    b = b0 * s
    idx = text.reshape(b).astype(jnp.int32)
    b_per_w = b // _NW
    out = _sc_gather(idx, letter_embedding, b_per_w, d)
    return out.reshape(b0, s, d)


# speedup vs baseline: 1.4550x; 1.0041x over previous
import functools

import jax
import jax.numpy as jnp
from jax import lax
from jax.experimental import pallas as pl
from jax.experimental.pallas import tpu as pltpu
from jax.experimental.pallas import tpu_sc as plsc

_NC = 2
_NS = 16
_NW = _NC * _NS

_CHUNK = 32
_LANES = 16
_STREAM_FRAC = 0.10


@functools.partial(jax.jit, static_argnums=(2, 3))
def _sc_gather(idx, table, b_per_w, d):
    b = idx.shape[0]
    v = table.shape[0]
    n_chunks = b_per_w // _CHUNK
    n_seg = d // _LANES
    a = int(n_chunks * _STREAM_FRAC) // 2 * 2
    nt = n_chunks - a
    ni = max(a, nt)
    assert ni % 2 == 0 and nt % 2 == 0
    mesh = plsc.VectorSubcoreMesh(core_axis_name="c", subcore_axis_name="s")

    @functools.partial(
        pl.kernel,
        out_type=jax.ShapeDtypeStruct((b, d), jnp.float32),
        mesh=mesh,
        scratch_types=[
            pltpu.VMEM((v, d), jnp.float32),
            pltpu.VMEM((b_per_w,), jnp.int32),
            pltpu.VMEM((_CHUNK, d), jnp.float32),
            pltpu.VMEM((_CHUNK, d), jnp.float32),
            pltpu.VMEM((_CHUNK, d), jnp.float32),
            pltpu.VMEM((_CHUNK, d), jnp.float32),
            pltpu.SemaphoreType.DMA,
            pltpu.SemaphoreType.DMA,
            pltpu.SemaphoreType.DMA,
            pltpu.SemaphoreType.DMA,
            pltpu.SemaphoreType.DMA,
            pltpu.SemaphoreType.DMA,
        ],
    )
    def k(idx_hbm, table_hbm, out_hbm, table_v, idx_v, sb0, sb1, tb0, tb1,
          gs0, gs1, ws0, ws1, wt0, wt1):
        wid = lax.axis_index("s") * _NC + lax.axis_index("c")
        base = wid * b_per_w
        sb = (sb0, sb1)
        tb = (tb0, tb1)
        gsem = (gs0, gs1)
        wssem = (ws0, ws1)
        wtsem = (wt0, wt1)

        pltpu.sync_copy(table_hbm, table_v)
        pltpu.sync_copy(idx_hbm.at[pl.ds(base, b_per_w)], idx_v)

        def g_copy(i, bi):
            return pltpu.make_async_copy(
                table_hbm.at[idx_v.at[pl.ds(i * _CHUNK, _CHUNK)]],
                sb[bi],
                gsem[bi],
            )

        def ws_copy(i, bi):
            return pltpu.make_async_copy(
                sb[bi],
                out_hbm.at[pl.ds(base + i * _CHUNK, _CHUNK)],
                wssem[bi],
            )

        def wt_copy(i, bi):
            return pltpu.make_async_copy(
                tb[bi],
                out_hbm.at[pl.ds(base + (a + i) * _CHUNK, _CHUNK)],
                wtsem[bi],
            )

        def fill(i, bi):
            def row_group(jj, _):
                j0 = jj * _LANES
                rvec = idx_v[pl.ds((a + i) * _CHUNK + j0, _LANES)]
                for j in range(_LANES):
                    r = rvec[j]
                    for s0 in range(0, n_seg, 4):
                        vals = [
                            table_v[r, pl.ds((s0 + s) * _LANES, _LANES)]
                            for s in range(4)
                        ]
                        for s in range(4):
                            tb[bi][
                                j0 + j, pl.ds((s0 + s) * _LANES, _LANES)
                            ] = vals[s]
                return 0

            lax.fori_loop(0, _CHUNK // _LANES, row_group, 0)

        if a >= 1:
            g_copy(0, 0).start()

        def step(i, bi):
            @pl.when(i < nt)
            def _():
                @pl.when(i >= 2)
                def _():
                    wt_copy(i - 2, bi).wait()

                fill(i, bi)
                wt_copy(i, bi).start()

            @pl.when(i < a)
            def _():
                @pl.when(i >= 1)
                def _():
                    ws_copy(i - 1, bi ^ 1).wait()

                @pl.when(i + 1 < a)
                def _():
                    g_copy(i + 1, bi ^ 1).start()

                g_copy(i, bi).wait()
                ws_copy(i, bi).start()

        def body(grp, _):
            i = grp * 2
            step(i, 0)
            step(i + 1, 1)
            return 0

        lax.fori_loop(0, ni // 2, body, 0)
        if a >= 1:
            ws_copy(a - 1, (a - 1) % 2).wait()
        wt_copy(nt - 2, 0).wait()
        wt_copy(nt - 1, 1).wait()

    return k(idx, table)


def kernel(text, max_seq_len, letter_embedding, positional_encoding):
    del max_seq_len, positional_encoding
    b0, s = text.shape
    v, d = letter_embedding.shape
    b = b0 * s
    idx = text.reshape(b).astype(jnp.int32)
    b_per_w = b // _NW
    out = _sc_gather(idx, letter_embedding, b_per_w, d)
    return out.reshape(b0, s, d)
